# baseline (speedup 1.0000x reference)
import functools

import jax
import jax.numpy as jnp
from jax import lax
from jax.experimental import pallas as pl
from jax.experimental.pallas import tpu as pltpu
from jax.experimental.pallas import tpu_sc as plsc

VOCAB = 1000000
D = 64
BATCH = 16384
EPS = 1e-10

NW = 32
ROWS_W = BATCH // NW
C = 64
NCHUNK = ROWS_W // C
NIDX = 5 * C
IDXPAD = NIDX + 64
GSUB = 64
NG = NIDX // GSUB
LANES = 16
PD = 128


def _rsqrt_nr(x):
    i = lax.bitcast_convert_type(x, jnp.int32)
    i = jnp.int32(0x5F3759DF) - lax.shift_right_logical(i, 1)
    y = lax.bitcast_convert_type(i, jnp.float32)
    for _ in range(3):
        y = y * (1.5 - 0.5 * x * y * y)
    return y


def _make_sc_kernel():
    mesh = plsc.VectorSubcoreMesh(core_axis_name="c", subcore_axis_name="s")

    @functools.partial(
        pl.kernel,
        mesh=mesh,
        compiler_params=pltpu.CompilerParams(
            needs_layout_passes=False, use_tc_tiling_on_sc=True),
        out_type=jax.ShapeDtypeStruct((BATCH * 3,), jnp.float32),
        scratch_types=[
            pltpu.VMEM((IDXPAD,), jnp.int32),
            pltpu.VMEM((IDXPAD,), jnp.int32),
            pltpu.VMEM((NIDX,), jnp.int32),
            pltpu.VMEM((NIDX,), jnp.int32),
            pltpu.VMEM((NIDX, PD), jnp.float32),
            pltpu.VMEM((NIDX, PD), jnp.float32),
            pltpu.VMEM((7 * C,), jnp.float32),
            pltpu.VMEM((3 * C, ), jnp.float32),
            pltpu.SemaphoreType.DMA,
            pltpu.SemaphoreType.DMA,
        ],
    )
    def sc_fn(idx_hbm, table_hbm, out_hbm, idx_v0, idx_v1, idxp0, idxp1,
              gbuf0, gbuf1, stats, outbuf, sem0, sem1):
        wid = lax.axis_index("s") * 2 + lax.axis_index("c")
        base_row = wid * ROWS_W
        sems = (sem0, sem1)
        idx_vs = (idx_v0, idx_v1)
        idxps = (idxp0, idxp1)
        gbufs = (gbuf0, gbuf1)

        def start_gather(t, slot):
            idx_v = idx_vs[slot]
            idxp = idxps[slot]
            off = (base_row + t * C) * 5
            pltpu.sync_copy(idx_hbm.at[pl.ds(off, NIDX)],
                            idx_v.at[pl.ds(0, NIDX)])
            for k in range(NIDX // LANES):
                v = idx_v[pl.ds(k * LANES, LANES)]
                idxp[pl.ds(k * LANES, LANES)] = (
                    lax.shift_right_logical(v, 1))
            handles = []
            for k in range(NG):
                handles.append(pltpu.async_copy(
                    table_hbm.at[idxp.at[pl.ds(k * GSUB, GSUB)]],
                    gbufs[slot].at[pl.ds(k * GSUB, GSUB)],
                    sems[slot]))
            return handles

        def compute_chunk(slot, t):
            lane = lax.iota(jnp.int32, LANES)
            m15 = lane == (LANES - 1)
            idx_v = idx_vs[slot]
            gbuf = gbufs[slot]

            def row(c, carry):
                r0 = 5 * c
                iv = idx_v[pl.ds(r0, LANES)]
                cb = jnp.full((LANES,), c, jnp.int32)

                def put(k, vec):
                    tot = jnp.cumsum(vec)
                    plsc.store_scatter(stats, [cb + (k * C)], tot, mask=m15)

                offx = (iv[0] & 1) * D
                offy = (iv[1] & 1) * D
                hv = []
                hh = None
                for j in range(D // LANES):
                    x = gbuf[r0, pl.ds(offx + LANES * j, LANES)]
                    y = gbuf[r0 + 1, pl.ds(offy + LANES * j, LANES)]
                    h = x - y
                    hv.append(h)
                    hh = h * h if hh is None else hh + h * h
                put(0, hh)
                for i in range(3):
                    offr = (iv[2 + i] & 1) * D
                    dv = None
                    rv = None
                    for j in range(D // LANES):
                        r = gbuf[r0 + 2 + i,
                                 pl.ds(offr + LANES * j, LANES)]
                        dv = hv[j] * r if dv is None else dv + hv[j] * r
                        rv = r * r if rv is None else rv + r * r
                    put(1 + i, dv)
                    put(4 + i, rv)
                return carry

            lax.fori_loop(0, C, row, 0)

            for g in range(C // LANES):
                hh = stats[pl.ds(LANES * g, LANES)]
                nh = jnp.maximum(hh * _rsqrt_nr(hh), EPS)
                ob = (g * LANES + lane) * 3
                for i in range(3):
                    dd = stats[pl.ds((1 + i) * C + LANES * g, LANES)]
                    rr = stats[pl.ds((4 + i) * C + LANES * g, LANES)]
                    nr = jnp.maximum(rr * _rsqrt_nr(rr), EPS)
                    plsc.store_scatter(outbuf, [ob + i], dd / (nh * nr))
            pltpu.sync_copy(
                outbuf, out_hbm.at[pl.ds((base_row + t * C) * 3, 3 * C)])

        handles = [None, None]
        handles[0] = start_gather(0, 0)
        for t in range(NCHUNK):
            slot = t % 2
            if t + 1 < NCHUNK:
                handles[1 - slot] = start_gather(t + 1, 1 - slot)
            for h in handles[slot]:
                h.wait()
            compute_chunk(slot, t)

    return sc_fn


_SC_KERNEL = _make_sc_kernel()


def kernel(input, onepole, four, table):
    idx_flat = input.astype(jnp.int32).reshape(-1)
    table2 = table.reshape(VOCAB // 2, 2 * D)
    out_flat = _SC_KERNEL(idx_flat, table2)
    return out_flat.reshape(BATCH, 3)


# speedup vs baseline: 1.1148x; 1.1148x over previous
import functools

import jax
import jax.numpy as jnp
from jax import lax
from jax.experimental import pallas as pl
from jax.experimental.pallas import tpu as pltpu
from jax.experimental.pallas import tpu_sc as plsc

VOCAB = 1000000
D = 64
BATCH = 16384
EPS = 1e-10

NW = 32
ROWS_W = BATCH // NW
C = 64
NCHUNK = ROWS_W // C
NIDX = 5 * C
IDXPAD = NIDX + 64
GSUB = 64
NG = NIDX // GSUB
LANES = 16
PD = 128


def _rsqrt_nr(x):
    i = lax.bitcast_convert_type(x, jnp.int32)
    i = jnp.int32(0x5F3759DF) - lax.shift_right_logical(i, 1)
    y = lax.bitcast_convert_type(i, jnp.float32)
    for _ in range(3):
        y = y * (1.5 - 0.5 * x * y * y)
    return y


def _make_sc_kernel():
    mesh = plsc.VectorSubcoreMesh(core_axis_name="c", subcore_axis_name="s")

    @functools.partial(
        pl.kernel,
        mesh=mesh,
        compiler_params=pltpu.CompilerParams(
            needs_layout_passes=False, use_tc_tiling_on_sc=True),
        out_type=jax.ShapeDtypeStruct((BATCH * 3,), jnp.float32),
        scratch_types=[
            pltpu.VMEM((IDXPAD,), jnp.int32),
            pltpu.VMEM((IDXPAD,), jnp.int32),
            pltpu.VMEM((NIDX, PD), jnp.float32),
            pltpu.VMEM((NIDX, PD), jnp.float32),
            pltpu.VMEM((7 * C,), jnp.float32),
            pltpu.VMEM((3 * C, ), jnp.float32),
            pltpu.SemaphoreType.DMA,
            pltpu.SemaphoreType.DMA,
        ],
    )
    def sc_fn(idx_hbm, table_hbm, out_hbm, idx_v0, idx_v1,
              gbuf0, gbuf1, stats, outbuf, sem0, sem1):
        wid = lax.axis_index("s") * 2 + lax.axis_index("c")
        base_row = wid * ROWS_W
        sems = (sem0, sem1)
        idx_vs = (idx_v0, idx_v1)
        gbufs = (gbuf0, gbuf1)

        def start_gather(t, slot):
            idx_v = idx_vs[slot]
            off = (base_row + t * C) * 5
            pltpu.sync_copy(idx_hbm.at[pl.ds(off, NIDX)],
                            idx_v.at[pl.ds(0, NIDX)])
            handles = []
            for k in range(NG):
                handles.append(pltpu.async_copy(
                    table_hbm.at[idx_v.at[pl.ds(k * GSUB, GSUB)]],
                    gbufs[slot].at[pl.ds(k * GSUB, GSUB)],
                    sems[slot]))
            return handles

        def compute_chunk(slot, t):
            lane = lax.iota(jnp.int32, LANES)
            m15 = lane == (LANES - 1)
            idx_v = idx_vs[slot]
            gbuf = gbufs[slot]

            def row(c, carry):
                r0 = 5 * c
                cb = jnp.full((LANES,), c, jnp.int32)

                def put(k, vec):
                    tot = jnp.cumsum(vec)
                    plsc.store_scatter(stats, [cb + (k * C)], tot, mask=m15)

                hv = []
                hh = None
                for j in range(D // LANES):
                    x = gbuf[r0, pl.ds(LANES * j, LANES)]
                    y = gbuf[r0 + 1, pl.ds(LANES * j, LANES)]
                    h = x - y
                    hv.append(h)
                    hh = h * h if hh is None else hh + h * h
                put(0, hh)
                for i in range(3):
                    dv = None
                    rv = None
                    for j in range(D // LANES):
                        r = gbuf[r0 + 2 + i, pl.ds(LANES * j, LANES)]
                        dv = hv[j] * r if dv is None else dv + hv[j] * r
                        rv = r * r if rv is None else rv + r * r
                    put(1 + i, dv)
                    put(4 + i, rv)
                return carry

            lax.fori_loop(0, C, row, 0)

            for g in range(C // LANES):
                hh = stats[pl.ds(LANES * g, LANES)]
                nh = jnp.maximum(hh * _rsqrt_nr(hh), EPS)
                ob = (g * LANES + lane) * 3
                for i in range(3):
                    dd = stats[pl.ds((1 + i) * C + LANES * g, LANES)]
                    rr = stats[pl.ds((4 + i) * C + LANES * g, LANES)]
                    nr = jnp.maximum(rr * _rsqrt_nr(rr), EPS)
                    plsc.store_scatter(outbuf, [ob + i], dd / (nh * nr))
            pltpu.sync_copy(
                outbuf, out_hbm.at[pl.ds((base_row + t * C) * 3, 3 * C)])

        handles = [None, None]
        handles[0] = start_gather(0, 0)
        for t in range(NCHUNK):
            slot = t % 2
            if t + 1 < NCHUNK:
                handles[1 - slot] = start_gather(t + 1, 1 - slot)
            for h in handles[slot]:
                h.wait()
            compute_chunk(slot, t)

    return sc_fn


_SC_KERNEL = _make_sc_kernel()


def kernel(input, onepole, four, table):
    idx_flat = input.astype(jnp.int32).reshape(-1)
    table2 = jnp.pad(table, ((0, 0), (0, PD - D)))
    out_flat = _SC_KERNEL(idx_flat, table2)
    return out_flat.reshape(BATCH, 3)


# speedup vs baseline: 1.3022x; 1.1681x over previous
import functools

import jax
import jax.numpy as jnp
from jax import lax
from jax.experimental import pallas as pl
from jax.experimental.pallas import tpu as pltpu
from jax.experimental.pallas import tpu_sc as plsc

VOCAB = 1000000
D = 64
BATCH = 16384
EPS = 1e-10

NW = 32
ROWS_W = BATCH // NW
CL = 8
NCHUNK = ROWS_W // CL
NFETCH = 5 * CL
NIDX = ROWS_W * 5
LANES = 16
GRP = 8


def _rsqrt_nr(x):
    i = lax.bitcast_convert_type(x, jnp.int32)
    i = jnp.int32(0x5F3759DF) - lax.shift_right_logical(i, 1)
    y = lax.bitcast_convert_type(i, jnp.float32)
    for _ in range(3):
        y = y * (1.5 - 0.5 * x * y * y)
    return y


def _make_sc_kernel():
    mesh = plsc.VectorSubcoreMesh(core_axis_name="c", subcore_axis_name="s")

    @functools.partial(
        pl.kernel,
        mesh=mesh,
        compiler_params=pltpu.CompilerParams(
            needs_layout_passes=False, use_tc_tiling_on_sc=True),
        out_type=jax.ShapeDtypeStruct((BATCH * 3,), jnp.float32),
        scratch_types=[
            pltpu.VMEM((NIDX + LANES,), jnp.int32),
            pltpu.VMEM((NFETCH, GRP, D), jnp.float32),
            pltpu.VMEM((NFETCH, GRP, D), jnp.float32),
            pltpu.VMEM((7 * 2 * CL,), jnp.float32),
            pltpu.VMEM((3 * 2 * CL,), jnp.float32),
            pltpu.SemaphoreType.DMA,
            pltpu.SemaphoreType.DMA,
        ],
    )
    def sc_fn(idx_hbm, table_hbm, out_hbm, idxall, gbuf0, gbuf1,
              stats, outbuf, sem0, sem1):
        wid = lax.axis_index("s") * 2 + lax.axis_index("c")
        base_row = wid * ROWS_W
        pltpu.sync_copy(idx_hbm.at[pl.ds(base_row * 5, NIDX)],
                        idxall.at[pl.ds(0, NIDX)])

        def issue(c, gb, sem):
            def jbody(j, carry):
                iv = idxall[pl.ds(c * NFETCH + 5 * j, LANES)]
                for o in range(5):
                    start = pl.multiple_of((iv[o] >> 3) * GRP, GRP)
                    pltpu.async_copy(
                        table_hbm.at[pl.ds(start, GRP), :],
                        gb.at[5 * j + o], sem)
                return carry
            lax.fori_loop(0, CL, jbody, 0)

        def drain(gb, sem):
            pltpu.make_async_copy(
                table_hbm.at[pl.ds(0, GRP), :], gb, sem).wait()

        def phase_a(c, gb, off):
            lane = lax.iota(jnp.int32, LANES)
            m15 = lane == (LANES - 1)

            def row(j, carry):
                iv = idxall[pl.ds(c * NFETCH + 5 * j, LANES)]
                cb = jnp.full((LANES,), j + off, jnp.int32)

                def put(k, vec):
                    tot = jnp.cumsum(vec)
                    plsc.store_scatter(
                        stats, [cb + (k * 2 * CL)], tot, mask=m15)

                rx = iv[0] & (GRP - 1)
                ry = iv[1] & (GRP - 1)
                hv = []
                hh = None
                for k in range(D // LANES):
                    x = gb[5 * j, rx, pl.ds(LANES * k, LANES)]
                    y = gb[5 * j + 1, ry, pl.ds(LANES * k, LANES)]
                    h = x - y
                    hv.append(h)
                    hh = h * h if hh is None else hh + h * h
                put(0, hh)
                for i in range(3):
                    rsel = iv[2 + i] & (GRP - 1)
                    dv = None
                    rv = None
                    for k in range(D // LANES):
                        r = gb[5 * j + 2 + i, rsel,
                               pl.ds(LANES * k, LANES)]
                        dv = hv[k] * r if dv is None else dv + hv[k] * r
                        rv = r * r if rv is None else rv + r * r
                    put(1 + i, dv)
                    put(4 + i, rv)
                return carry

            lax.fori_loop(0, CL, row, 0)

        def phase_b(tp):
            lane = lax.iota(jnp.int32, LANES)
            hh = stats[pl.ds(0, LANES)]
            nh = jnp.maximum(hh * _rsqrt_nr(hh), EPS)
            ob = lane * 3
            for i in range(3):
                dd = stats[pl.ds((1 + i) * 2 * CL, LANES)]
                rr = stats[pl.ds((4 + i) * 2 * CL, LANES)]
                nr = jnp.maximum(rr * _rsqrt_nr(rr), EPS)
                plsc.store_scatter(outbuf, [ob + i], dd / (nh * nr))
            pltpu.sync_copy(
                outbuf,
                out_hbm.at[pl.ds((base_row + tp * 2 * CL) * 3, 6 * CL)])

        issue(0, gbuf0, sem0)

        def pair(tp, carry):
            c0 = 2 * tp
            issue(c0 + 1, gbuf1, sem1)
            drain(gbuf0, sem0)
            phase_a(c0, gbuf0, 0)

            @pl.when(c0 + 2 < NCHUNK)
            def _():
                issue(c0 + 2, gbuf0, sem0)

            drain(gbuf1, sem1)
            phase_a(c0 + 1, gbuf1, CL)
            phase_b(tp)
            return carry

        lax.fori_loop(0, NCHUNK // 2, pair, 0)

    return sc_fn


_SC_KERNEL = _make_sc_kernel()


def kernel(input, onepole, four, table):
    idx_flat = input.astype(jnp.int32).reshape(-1)
    out_flat = _SC_KERNEL(idx_flat, table)
    return out_flat.reshape(BATCH, 3)
